# Initial kernel scaffold; baseline (speedup 1.0000x reference)
import jax
import jax.numpy as jnp
from jax.experimental import pallas as pl


def kernel(compound_diseases, compound_phenotypes, compound_subcellular_locations, protein_diseases, protein_phenotypes, protein_subcellular_locations, disease_table, phenotype_table, sub_table, W1, b1, W2, b2, W3, b3):
    raise NotImplementedError("write your pallas kernel here")


# speedup vs baseline: 13.5860x; 13.5860x over previous
import functools

import jax
import jax.numpy as jnp
from jax import lax
from jax.experimental import pallas as pl
from jax.experimental.pallas import tpu as pltpu
from jax.experimental.pallas import tpu_sc as plsc

B = 16384
L = 200
LS = 20
DD, DP, DS = 32, 16, 16
FEAT = (DD + DP + DS) * 2
H1, H2 = 128, 64

NC, NS = 2, 16
NW = NC * NS
RPW = B // NW
CB = 16
NBLK = RPW // CB


def _gather_rows(tab_hbm, idx_ref, off, n, dst):
    idx_slice = idx_ref.at[pl.ds(pl.multiple_of(off, 8), n)]
    pltpu.sync_copy(tab_hbm.at[idx_slice], dst)


def _sc_featurize_body(cd, cp, cs, pd, pp, ps, dis_t, phe_t, sub_t, out_hbm,
                       idx_v, rows32, rows16, stage, sem_a, sem_b):
    wid = lax.axis_index("s") * NC + lax.axis_index("c")
    wrow = wid * RPW

    def long_pass(idx_hbm, tab_hbm, d, col):
        wbase = wid * (RPW * L)
        rows = rows32 if d == 32 else rows16
        inv = 1.0 / L

        def block_body(blk, _):
            boff = pl.multiple_of(wbase + blk * (CB * L), 8)
            pltpu.sync_copy(idx_hbm.at[pl.ds(boff, CB * L)],
                            idx_v.at[pl.ds(0, CB * L)])

            def row_body(rb, _):
                o = rb * L
                _gather_rows(tab_hbm, idx_v, o, 128, rows.at[0, pl.ds(0, 128), :])
                _gather_rows(tab_hbm, idx_v, o + 128, 72, rows.at[0, pl.ds(128, 72), :])

                if d == 32:
                    def red(j, acc):
                        a0, a1, b0, b1 = acc
                        base = j * 8
                        for t in range(0, 8, 2):
                            r0 = base + t
                            a0 = a0 + rows[0, r0, pl.ds(0, 16)]
                            a1 = a1 + rows[0, r0, pl.ds(16, 16)]
                            b0 = b0 + rows[0, r0 + 1, pl.ds(0, 16)]
                            b1 = b1 + rows[0, r0 + 1, pl.ds(16, 16)]
                        return a0, a1, b0, b1
                    z = jnp.zeros((16,), jnp.float32)
                    a0, a1, b0, b1 = lax.fori_loop(0, 25, red, (z, z, z, z))
                    row = blk * CB + rb
                    stage[row, pl.ds(0, 16)] = (a0 + b0) * inv
                    stage[row, pl.ds(16, 16)] = (a1 + b1) * inv
                else:
                    def red(j, acc):
                        a0, b0 = acc
                        base = j * 8
                        for t in range(0, 8, 2):
                            r0 = base + t
                            a0 = a0 + rows[0, r0, pl.ds(0, 16)]
                            b0 = b0 + rows[0, r0 + 1, pl.ds(0, 16)]
                        return a0, b0
                    z = jnp.zeros((16,), jnp.float32)
                    a0, b0 = lax.fori_loop(0, 25, red, (z, z))
                    row = blk * CB + rb
                    stage[row, pl.ds(0, 16)] = (a0 + b0) * inv
                return 0

            lax.fori_loop(0, CB, row_body, 0)
            return 0

        lax.fori_loop(0, NBLK, block_body, 0)
        src = stage if d == 32 else stage.at[:, pl.ds(0, 16)]
        pltpu.sync_copy(src, out_hbm.at[pl.ds(wrow, RPW), pl.ds(col, d)])

    def sub_pass(idx_hbm, col):
        wbase = wid * (RPW * LS)
        inv = 1.0 / LS
        pltpu.sync_copy(idx_hbm.at[pl.ds(pl.multiple_of(wbase, 8), RPW * LS)],
                        idx_v.at[pl.ds(0, RPW * LS)])

        def chunk_body(c, _):
            o = c * 80
            _gather_rows(sub_t, idx_v, o, 80, rows16.at[0, pl.ds(0, 80), :])
            for seg in range(4):
                acc = jnp.zeros((16,), jnp.float32)
                for j in range(LS):
                    acc = acc + rows16[0, seg * LS + j, pl.ds(0, 16)]
                stage[c * 4 + seg, pl.ds(0, 16)] = acc * inv
            return 0

        lax.fori_loop(0, RPW // 4, chunk_body, 0)
        pltpu.sync_copy(stage.at[:, pl.ds(0, 16)],
                        out_hbm.at[pl.ds(wrow, RPW), pl.ds(col, 16)])

    long_pass(cd, dis_t, 32, 0)
    long_pass(cp, phe_t, 16, 32)
    sub_pass(cs, 48)
    long_pass(pd, dis_t, 32, 64)
    long_pass(pp, phe_t, 16, 96)
    sub_pass(ps, 112)


def _sc_featurize(cd, cp, cs, pd, pp, ps, dis_t, phe_t, sub_t):
    mesh = plsc.VectorSubcoreMesh(core_axis_name="c", subcore_axis_name="s")
    f = pl.kernel(
        _sc_featurize_body,
        out_type=jax.ShapeDtypeStruct((B, FEAT), jnp.float32),
        mesh=mesh,
        compiler_params=pltpu.CompilerParams(use_tc_tiling_on_sc=False),
        scratch_types=[
            pltpu.VMEM((RPW * LS,), jnp.int32),
            pltpu.VMEM((2, L, 32), jnp.float32),
            pltpu.VMEM((2, L, 16), jnp.float32),
            pltpu.VMEM((RPW, 32), jnp.float32),
            pltpu.SemaphoreType.DMA,
            pltpu.SemaphoreType.DMA,
        ],
    )
    return f(cd, cp, cs, pd, pp, ps, dis_t, phe_t, sub_t)


def _mlp_body(x_ref, w1_ref, b1_ref, w2_ref, b2_ref, w3_ref, b3_ref, o_ref):
    x = x_ref[...]
    h = jnp.dot(x, w1_ref[...], preferred_element_type=jnp.float32)
    h = h + b1_ref[...]
    h = jnp.where(h >= 0, h, 0.01 * h)
    h = jnp.dot(h, w2_ref[...], preferred_element_type=jnp.float32)
    h = h + b2_ref[...]
    h = jnp.where(h >= 0, h, 0.01 * h)
    o = jnp.dot(h, w3_ref[...], preferred_element_type=jnp.float32)
    o_ref[...] = o + b3_ref[...]


def _mlp(x, W1, b1, W2, b2, W3, b3):
    BB = 1024
    return pl.pallas_call(
        _mlp_body,
        grid=(B // BB,),
        in_specs=[
            pl.BlockSpec((BB, FEAT), lambda i: (i, 0)),
            pl.BlockSpec((FEAT, H1), lambda i: (0, 0)),
            pl.BlockSpec((1, H1), lambda i: (0, 0)),
            pl.BlockSpec((H1, H2), lambda i: (0, 0)),
            pl.BlockSpec((1, H2), lambda i: (0, 0)),
            pl.BlockSpec((H2, 1), lambda i: (0, 0)),
            pl.BlockSpec((1, 1), lambda i: (0, 0)),
        ],
        out_specs=pl.BlockSpec((BB, 1), lambda i: (i, 0)),
        out_shape=jax.ShapeDtypeStruct((B, 1), jnp.float32),
    )(x, W1, b1.reshape(1, H1), W2, b2.reshape(1, H2), W3, b3.reshape(1, 1))


def kernel(compound_diseases, compound_phenotypes, compound_subcellular_locations,
           protein_diseases, protein_phenotypes, protein_subcellular_locations,
           disease_table, phenotype_table, sub_table, W1, b1, W2, b2, W3, b3):
    cd = compound_diseases.reshape(-1).astype(jnp.int32)
    cp = compound_phenotypes.reshape(-1).astype(jnp.int32)
    cs = compound_subcellular_locations.reshape(-1).astype(jnp.int32)
    pd = protein_diseases.reshape(-1).astype(jnp.int32)
    pp = protein_phenotypes.reshape(-1).astype(jnp.int32)
    ps = protein_subcellular_locations.reshape(-1).astype(jnp.int32)
    x = _sc_featurize(cd, cp, cs, pd, pp, ps, disease_table, phenotype_table, sub_table)
    return _mlp(x, W1, b1, W2, b2, W3, b3)


# speedup vs baseline: 32.9778x; 2.4273x over previous
import jax
import jax.numpy as jnp
from jax import lax
from jax.experimental import pallas as pl
from jax.experimental.pallas import tpu as pltpu
from jax.experimental.pallas import tpu_sc as plsc

B = 16384
L = 200
LS = 20
DD, DP, DS = 32, 16, 16
FEAT = (DD + DP + DS) * 2
H1, H2 = 128, 64

NC, NS = 2, 16
NW = NC * NS
RPW = B // NW
SB = 64
NSB = RPW // SB
NB = 4
K = 3


def _sc_featurize_body(cd, cp, cs, pd, pp, ps, dis_t, phe_t, sub_t, out_hbm,
                       idx_v, rows32, rows16, stage, *sems):
    wid = lax.axis_index("s") * NC + lax.axis_index("c")
    wrow = wid * RPW

    def fire(tab_hbm, rows, buf, off, sem):
        o = pl.multiple_of(off, 8)
        pltpu.async_copy(tab_hbm.at[idx_v.at[pl.ds(o, 128)]],
                         rows.at[buf, pl.ds(0, 128), :], sem)
        o2 = pl.multiple_of(off + 128, 8)
        pltpu.async_copy(tab_hbm.at[idx_v.at[pl.ds(o2, 72)]],
                         rows.at[buf, pl.ds(128, 72), :], sem)

    def drain(tab_hbm, rows, buf, sem):
        pltpu.make_async_copy(tab_hbm.at[idx_v.at[pl.ds(0, 128)]],
                              rows.at[buf, pl.ds(0, 128), :], sem).wait()
        pltpu.make_async_copy(tab_hbm.at[idx_v.at[pl.ds(0, 72)]],
                              rows.at[buf, pl.ds(128, 72), :], sem).wait()

    def long_pass(idx_hbm, tab_hbm, d, col):
        wbase = wid * (RPW * L)
        rows = rows32 if d == 32 else rows16
        inv = 1.0 / L

        def sblock_body(sb, _):
            boff = pl.multiple_of(wbase + sb * (SB * L), 8)
            pltpu.sync_copy(idx_hbm.at[pl.ds(boff, SB * L)],
                            idx_v.at[pl.ds(0, SB * L)])
            for p in range(K):
                fire(tab_hbm, rows, p % NB, p * L, sems[p % NB])

            def quad_body(q, _):
                for i in range(4):
                    cur = q * 4 + i
                    fpos = cur + K
                    fbuf = (i + K) % NB

                    @pl.when(fpos < SB)
                    def _():
                        fire(tab_hbm, rows, fbuf, fpos * L, sems[fbuf])

                    drain(tab_hbm, rows, i, sems[i])

                    if d == 32:
                        def red(j, acc):
                            a0, a1, b0, b1 = acc
                            base = j * 8
                            for t in range(0, 8, 2):
                                r0 = base + t
                                a0 = a0 + rows[i, r0, pl.ds(0, 16)]
                                a1 = a1 + rows[i, r0, pl.ds(16, 16)]
                                b0 = b0 + rows[i, r0 + 1, pl.ds(0, 16)]
                                b1 = b1 + rows[i, r0 + 1, pl.ds(16, 16)]
                            return a0, a1, b0, b1
                        z = jnp.zeros((16,), jnp.float32)
                        a0, a1, b0, b1 = lax.fori_loop(0, 25, red, (z, z, z, z))
                        row = sb * SB + cur
                        stage[row, pl.ds(0, 16)] = (a0 + b0) * inv
                        stage[row, pl.ds(16, 16)] = (a1 + b1) * inv
                    else:
                        def red(j, acc):
                            a0, b0 = acc
                            base = j * 8
                            for t in range(0, 8, 2):
                                r0 = base + t
                                a0 = a0 + rows[i, r0, pl.ds(0, 16)]
                                b0 = b0 + rows[i, r0 + 1, pl.ds(0, 16)]
                            return a0, b0
                        z = jnp.zeros((16,), jnp.float32)
                        a0, b0 = lax.fori_loop(0, 25, red, (z, z))
                        row = sb * SB + cur
                        stage[row, pl.ds(0, 16)] = (a0 + b0) * inv
                return 0

            lax.fori_loop(0, SB // 4, quad_body, 0)
            return 0

        lax.fori_loop(0, NSB, sblock_body, 0)
        src = stage if d == 32 else stage.at[:, pl.ds(0, 16)]
        pltpu.sync_copy(src, out_hbm.at[pl.ds(wrow, RPW), pl.ds(col, d)])

    def fire_s(buf, off, sem):
        o = pl.multiple_of(off, 8)
        pltpu.async_copy(sub_t.at[idx_v.at[pl.ds(o, 80)]],
                         rows16.at[buf, pl.ds(0, 80), :], sem)

    def drain_s(buf, sem):
        pltpu.make_async_copy(sub_t.at[idx_v.at[pl.ds(0, 80)]],
                              rows16.at[buf, pl.ds(0, 80), :], sem).wait()

    def sub_pass(idx_hbm, col):
        wbase = wid * (RPW * LS)
        inv = 1.0 / LS
        nchunks = RPW // 4
        pltpu.sync_copy(idx_hbm.at[pl.ds(pl.multiple_of(wbase, 8), RPW * LS)],
                        idx_v.at[pl.ds(0, RPW * LS)])
        for p in range(2):
            fire_s(p % NB, p * 80, sems[p % NB])

        def quad_body(q, _):
            for i in range(4):
                c = q * 4 + i
                fc = c + 2
                fbuf = (i + 2) % NB

                @pl.when(fc < nchunks)
                def _():
                    fire_s(fbuf, fc * 80, sems[fbuf])

                drain_s(i, sems[i])
                for seg in range(4):
                    acc = jnp.zeros((16,), jnp.float32)
                    for j in range(LS):
                        acc = acc + rows16[i, seg * LS + j, pl.ds(0, 16)]
                    stage[c * 4 + seg, pl.ds(0, 16)] = acc * inv
            return 0

        lax.fori_loop(0, nchunks // 4, quad_body, 0)
        pltpu.sync_copy(stage.at[:, pl.ds(0, 16)],
                        out_hbm.at[pl.ds(wrow, RPW), pl.ds(col, 16)])

    long_pass(cd, dis_t, 32, 0)
    long_pass(cp, phe_t, 16, 32)
    sub_pass(cs, 48)
    long_pass(pd, dis_t, 32, 64)
    long_pass(pp, phe_t, 16, 96)
    sub_pass(ps, 112)


def _sc_featurize(cd, cp, cs, pd, pp, ps, dis_t, phe_t, sub_t):
    mesh = plsc.VectorSubcoreMesh(core_axis_name="c", subcore_axis_name="s")
    f = pl.kernel(
        _sc_featurize_body,
        out_type=jax.ShapeDtypeStruct((B, FEAT), jnp.float32),
        mesh=mesh,
        compiler_params=pltpu.CompilerParams(use_tc_tiling_on_sc=False),
        scratch_types=[
            pltpu.VMEM((SB * L,), jnp.int32),
            pltpu.VMEM((NB, L, 32), jnp.float32),
            pltpu.VMEM((NB, L, 16), jnp.float32),
            pltpu.VMEM((RPW, 32), jnp.float32),
        ] + [pltpu.SemaphoreType.DMA] * NB,
    )
    return f(cd, cp, cs, pd, pp, ps, dis_t, phe_t, sub_t)


def _mlp_body(x_ref, w1_ref, b1_ref, w2_ref, b2_ref, w3_ref, b3_ref, o_ref):
    x = x_ref[...]
    h = jnp.dot(x, w1_ref[...], preferred_element_type=jnp.float32)
    h = h + b1_ref[...]
    h = jnp.where(h >= 0, h, 0.01 * h)
    h = jnp.dot(h, w2_ref[...], preferred_element_type=jnp.float32)
    h = h + b2_ref[...]
    h = jnp.where(h >= 0, h, 0.01 * h)
    o = jnp.dot(h, w3_ref[...], preferred_element_type=jnp.float32)
    o_ref[...] = o + b3_ref[...]


def _mlp(x, W1, b1, W2, b2, W3, b3):
    BB = 1024
    return pl.pallas_call(
        _mlp_body,
        grid=(B // BB,),
        in_specs=[
            pl.BlockSpec((BB, FEAT), lambda i: (i, 0)),
            pl.BlockSpec((FEAT, H1), lambda i: (0, 0)),
            pl.BlockSpec((1, H1), lambda i: (0, 0)),
            pl.BlockSpec((H1, H2), lambda i: (0, 0)),
            pl.BlockSpec((1, H2), lambda i: (0, 0)),
            pl.BlockSpec((H2, 1), lambda i: (0, 0)),
            pl.BlockSpec((1, 1), lambda i: (0, 0)),
        ],
        out_specs=pl.BlockSpec((BB, 1), lambda i: (i, 0)),
        out_shape=jax.ShapeDtypeStruct((B, 1), jnp.float32),
    )(x, W1, b1.reshape(1, H1), W2, b2.reshape(1, H2), W3, b3.reshape(1, 1))


def kernel(compound_diseases, compound_phenotypes, compound_subcellular_locations,
           protein_diseases, protein_phenotypes, protein_subcellular_locations,
           disease_table, phenotype_table, sub_table, W1, b1, W2, b2, W3, b3):
    cd = compound_diseases.reshape(-1).astype(jnp.int32)
    cp = compound_phenotypes.reshape(-1).astype(jnp.int32)
    cs = compound_subcellular_locations.reshape(-1).astype(jnp.int32)
    pd = protein_diseases.reshape(-1).astype(jnp.int32)
    pp = protein_phenotypes.reshape(-1).astype(jnp.int32)
    ps = protein_subcellular_locations.reshape(-1).astype(jnp.int32)
    x = _sc_featurize(cd, cp, cs, pd, pp, ps, disease_table, phenotype_table, sub_table)
    return _mlp(x, W1, b1, W2, b2, W3, b3)


# speedup vs baseline: 81.1447x; 2.4606x over previous
import jax
import jax.numpy as jnp
from jax import lax
from jax.experimental import pallas as pl
from jax.experimental.pallas import tpu as pltpu
from jax.experimental.pallas import tpu_sc as plsc

B = 16384
L = 200
LS = 20
DD, DP, DS = 32, 16, 16
FEAT = (DD + DP + DS) * 2
H1, H2 = 128, 64

NC, NS = 2, 16
NW = NC * NS
RPW = B // NW
SB = 64
NSB = RPW // SB
NB = 4
K = 3


def _sc_featurize_body(cd, cp, cs, pd, pp, ps, dis_hbm, phe_hbm, sub_hbm, out_hbm,
                       dis_t, phe_t, sub_t, idx_v, rows32, rows16, stage, *sems):
    wid = lax.axis_index("s") * NC + lax.axis_index("c")
    wrow = wid * RPW
    sid = lax.axis_index("s")

    nd = dis_hbm.shape[0] // NS
    np_ = phe_hbm.shape[0] // NS
    pltpu.sync_copy(dis_hbm.at[pl.ds(sid * nd, nd), :], dis_t.at[pl.ds(sid * nd, nd), :])
    pltpu.sync_copy(phe_hbm.at[pl.ds(sid * np_, np_), :], phe_t.at[pl.ds(sid * np_, np_), :])

    @pl.when(sid == 0)
    def _():
        rem_d = dis_hbm.shape[0] - nd * NS
        rem_p = phe_hbm.shape[0] - np_ * NS
        pltpu.sync_copy(dis_hbm.at[pl.ds(nd * NS, rem_d), :], dis_t.at[pl.ds(nd * NS, rem_d), :])
        pltpu.sync_copy(phe_hbm.at[pl.ds(np_ * NS, rem_p), :], phe_t.at[pl.ds(np_ * NS, rem_p), :])
        pltpu.sync_copy(sub_hbm, sub_t)

    plsc.subcore_barrier()

    def fire(tab_hbm, rows, buf, off, sem):
        o = pl.multiple_of(off, 8)
        pltpu.async_copy(tab_hbm.at[idx_v.at[pl.ds(o, 128)]],
                         rows.at[buf, pl.ds(0, 128), :], sem)
        o2 = pl.multiple_of(off + 128, 8)
        pltpu.async_copy(tab_hbm.at[idx_v.at[pl.ds(o2, 72)]],
                         rows.at[buf, pl.ds(128, 72), :], sem)

    def drain(tab_hbm, rows, buf, sem):
        pltpu.make_async_copy(tab_hbm.at[idx_v.at[pl.ds(0, 128)]],
                              rows.at[buf, pl.ds(0, 128), :], sem).wait()
        pltpu.make_async_copy(tab_hbm.at[idx_v.at[pl.ds(0, 72)]],
                              rows.at[buf, pl.ds(128, 72), :], sem).wait()

    def long_pass(idx_hbm, tab_hbm, d, col):
        wbase = wid * (RPW * L)
        rows = rows32 if d == 32 else rows16
        inv = 1.0 / L

        def sblock_body(sb, _):
            boff = pl.multiple_of(wbase + sb * (SB * L), 8)
            pltpu.sync_copy(idx_hbm.at[pl.ds(boff, SB * L)],
                            idx_v.at[pl.ds(0, SB * L)])
            for p in range(K):
                fire(tab_hbm, rows, p % NB, p * L, sems[p % NB])

            def quad_body(q, _):
                for i in range(4):
                    cur = q * 4 + i
                    fpos = cur + K
                    fbuf = (i + K) % NB

                    @pl.when(fpos < SB)
                    def _():
                        fire(tab_hbm, rows, fbuf, fpos * L, sems[fbuf])

                    drain(tab_hbm, rows, i, sems[i])

                    if d == 32:
                        def red(j, acc):
                            a0, a1, b0, b1 = acc
                            base = j * 8
                            for t in range(0, 8, 2):
                                r0 = base + t
                                a0 = a0 + rows[i, r0, pl.ds(0, 16)]
                                a1 = a1 + rows[i, r0, pl.ds(16, 16)]
                                b0 = b0 + rows[i, r0 + 1, pl.ds(0, 16)]
                                b1 = b1 + rows[i, r0 + 1, pl.ds(16, 16)]
                            return a0, a1, b0, b1
                        z = jnp.zeros((16,), jnp.float32)
                        a0, a1, b0, b1 = lax.fori_loop(0, 25, red, (z, z, z, z))
                        row = sb * SB + cur
                        stage[row, pl.ds(0, 16)] = (a0 + b0) * inv
                        stage[row, pl.ds(16, 16)] = (a1 + b1) * inv
                    else:
                        def red(j, acc):
                            a0, b0 = acc
                            base = j * 8
                            for t in range(0, 8, 2):
                                r0 = base + t
                                a0 = a0 + rows[i, r0, pl.ds(0, 16)]
                                b0 = b0 + rows[i, r0 + 1, pl.ds(0, 16)]
                            return a0, b0
                        z = jnp.zeros((16,), jnp.float32)
                        a0, b0 = lax.fori_loop(0, 25, red, (z, z))
                        row = sb * SB + cur
                        stage[row, pl.ds(0, 16)] = (a0 + b0) * inv
                return 0

            lax.fori_loop(0, SB // 4, quad_body, 0)
            return 0

        lax.fori_loop(0, NSB, sblock_body, 0)
        src = stage if d == 32 else stage.at[:, pl.ds(0, 16)]
        pltpu.sync_copy(src, out_hbm.at[pl.ds(wrow, RPW), pl.ds(col, d)])

    def fire_s(buf, off, sem):
        o = pl.multiple_of(off, 8)
        pltpu.async_copy(sub_t.at[idx_v.at[pl.ds(o, 80)]],
                         rows16.at[buf, pl.ds(0, 80), :], sem)

    def drain_s(buf, sem):
        pltpu.make_async_copy(sub_t.at[idx_v.at[pl.ds(0, 80)]],
                              rows16.at[buf, pl.ds(0, 80), :], sem).wait()

    def sub_pass(idx_hbm, col):
        wbase = wid * (RPW * LS)
        inv = 1.0 / LS
        nchunks = RPW // 4
        pltpu.sync_copy(idx_hbm.at[pl.ds(pl.multiple_of(wbase, 8), RPW * LS)],
                        idx_v.at[pl.ds(0, RPW * LS)])
        for p in range(2):
            fire_s(p % NB, p * 80, sems[p % NB])

        def quad_body(q, _):
            for i in range(4):
                c = q * 4 + i
                fc = c + 2
                fbuf = (i + 2) % NB

                @pl.when(fc < nchunks)
                def _():
                    fire_s(fbuf, fc * 80, sems[fbuf])

                drain_s(i, sems[i])
                for seg in range(4):
                    acc = jnp.zeros((16,), jnp.float32)
                    for j in range(LS):
                        acc = acc + rows16[i, seg * LS + j, pl.ds(0, 16)]
                    stage[c * 4 + seg, pl.ds(0, 16)] = acc * inv
            return 0

        lax.fori_loop(0, nchunks // 4, quad_body, 0)
        pltpu.sync_copy(stage.at[:, pl.ds(0, 16)],
                        out_hbm.at[pl.ds(wrow, RPW), pl.ds(col, 16)])

    long_pass(cd, dis_t, 32, 0)
    long_pass(cp, phe_t, 16, 32)
    sub_pass(cs, 48)
    long_pass(pd, dis_t, 32, 64)
    long_pass(pp, phe_t, 16, 96)
    sub_pass(ps, 112)


def _sc_featurize(cd, cp, cs, pd, pp, ps, dis_t, phe_t, sub_t):
    mesh = plsc.VectorSubcoreMesh(core_axis_name="c", subcore_axis_name="s")
    f = pl.kernel(
        _sc_featurize_body,
        out_type=jax.ShapeDtypeStruct((B, FEAT), jnp.float32),
        mesh=mesh,
        compiler_params=pltpu.CompilerParams(use_tc_tiling_on_sc=False),
        scratch_types=[
            pltpu.VMEM_SHARED((13752, 32), jnp.float32),
            pltpu.VMEM_SHARED((17393, 16), jnp.float32),
            pltpu.VMEM_SHARED((30, 16), jnp.float32),
            pltpu.VMEM((SB * L,), jnp.int32),
            pltpu.VMEM((NB, L, 32), jnp.float32),
            pltpu.VMEM((NB, L, 16), jnp.float32),
            pltpu.VMEM((RPW, 32), jnp.float32),
        ] + [pltpu.SemaphoreType.DMA] * NB,
    )
    return f(cd, cp, cs, pd, pp, ps, dis_t, phe_t, sub_t)


def _mlp_body(x_ref, w1_ref, b1_ref, w2_ref, b2_ref, w3_ref, b3_ref, o_ref):
    x = x_ref[...]
    h = jnp.dot(x, w1_ref[...], preferred_element_type=jnp.float32)
    h = h + b1_ref[...]
    h = jnp.where(h >= 0, h, 0.01 * h)
    h = jnp.dot(h, w2_ref[...], preferred_element_type=jnp.float32)
    h = h + b2_ref[...]
    h = jnp.where(h >= 0, h, 0.01 * h)
    o = jnp.dot(h, w3_ref[...], preferred_element_type=jnp.float32)
    o_ref[...] = o + b3_ref[...]


def _mlp(x, W1, b1, W2, b2, W3, b3):
    BB = 1024
    return pl.pallas_call(
        _mlp_body,
        grid=(B // BB,),
        in_specs=[
            pl.BlockSpec((BB, FEAT), lambda i: (i, 0)),
            pl.BlockSpec((FEAT, H1), lambda i: (0, 0)),
            pl.BlockSpec((1, H1), lambda i: (0, 0)),
            pl.BlockSpec((H1, H2), lambda i: (0, 0)),
            pl.BlockSpec((1, H2), lambda i: (0, 0)),
            pl.BlockSpec((H2, 1), lambda i: (0, 0)),
            pl.BlockSpec((1, 1), lambda i: (0, 0)),
        ],
        out_specs=pl.BlockSpec((BB, 1), lambda i: (i, 0)),
        out_shape=jax.ShapeDtypeStruct((B, 1), jnp.float32),
    )(x, W1, b1.reshape(1, H1), W2, b2.reshape(1, H2), W3, b3.reshape(1, 1))


def kernel(compound_diseases, compound_phenotypes, compound_subcellular_locations,
           protein_diseases, protein_phenotypes, protein_subcellular_locations,
           disease_table, phenotype_table, sub_table, W1, b1, W2, b2, W3, b3):
    cd = compound_diseases.reshape(-1).astype(jnp.int32)
    cp = compound_phenotypes.reshape(-1).astype(jnp.int32)
    cs = compound_subcellular_locations.reshape(-1).astype(jnp.int32)
    pd = protein_diseases.reshape(-1).astype(jnp.int32)
    pp = protein_phenotypes.reshape(-1).astype(jnp.int32)
    ps = protein_subcellular_locations.reshape(-1).astype(jnp.int32)
    x = _sc_featurize(cd, cp, cs, pd, pp, ps, disease_table, phenotype_table, sub_table)
    return _mlp(x, W1, b1, W2, b2, W3, b3)
